# baseline (speedup 1.0000x reference)
import jax
import jax.numpy as jnp
from jax import lax
from jax.experimental import pallas as pl
from jax.experimental.pallas import tpu as pltpu
from jax.experimental.pallas import tpu_sc as plsc

EMBED_DIM = 64
NUM_BATCHES = 1000
B = 16384

NC = 2
NS = 16
NW = NC * NS
B_PER_W = B // NW
N_SUB = 4
SUB = B_PER_W // N_SUB
ROWS2 = B_PER_W // 2


def _sc_body(x_hbm, idx_hbm, table_hbm, out_hbm,
             idx_v, rows_v, x_v, x_sem, g_sems, o_sem):
    wid = lax.axis_index("s") * NC + lax.axis_index("c")
    base = wid * ROWS2

    x_copy = pltpu.async_copy(x_hbm.at[pl.ds(base, ROWS2)], x_v, x_sem)
    pltpu.sync_copy(idx_hbm.at[wid], idx_v)
    gathers = [
        pltpu.async_copy(table_hbm.at[idx_v.at[j]], rows_v.at[j], g_sems.at[j])
        for j in range(N_SUB)
    ]
    x_copy.wait()

    stores = []
    for j in range(N_SUB):
        gathers[j].wait()

        def sub_pair(q, _):
            R = j * (SUB // 2) + q
            for h in range(2):
                for c in range(EMBED_DIM // 16):
                    sl = pl.ds(h * EMBED_DIM + c * 16, 16)
                    rl = pl.ds(c * 16, 16)
                    x_v[R, sl] = x_v[R, sl] - rows_v[j, 2 * q + h, rl]
            return 0

        lax.fori_loop(0, SUB // 2, sub_pair, 0)
        stores.append(pltpu.async_copy(
            x_v.at[pl.ds(j * (SUB // 2), SUB // 2)],
            out_hbm.at[pl.ds(base + j * (SUB // 2), SUB // 2)], o_sem))
    for s in stores:
        s.wait()


@jax.jit
def _batch_correct(x, batch_labels, batch_embed):
    mesh = plsc.VectorSubcoreMesh(core_axis_name="c", subcore_axis_name="s")
    x2 = x.reshape(B // 2, 2 * EMBED_DIM)
    idx3 = batch_labels.reshape(NW, N_SUB, SUB)
    out2 = pl.kernel(
        _sc_body,
        out_type=jax.ShapeDtypeStruct((B // 2, 2 * EMBED_DIM), jnp.float32),
        mesh=mesh,
        scratch_types=[
            pltpu.VMEM((N_SUB, SUB), jnp.int32),
            pltpu.VMEM((N_SUB, SUB, EMBED_DIM), jnp.float32),
            pltpu.VMEM((ROWS2, 2 * EMBED_DIM), jnp.float32),
            pltpu.SemaphoreType.DMA,
            pltpu.SemaphoreType.DMA((N_SUB,)),
            pltpu.SemaphoreType.DMA,
        ],
        compiler_params=pltpu.CompilerParams(
            use_tc_tiling_on_sc=False,
            disable_bounds_checks=True,
            disable_semaphore_checks=True,
            skip_device_barrier=True,
        ),
    )(x2, idx3, batch_embed)
    return out2.reshape(B, EMBED_DIM)


def kernel(x, batch_labels, batch_embed):
    return _batch_correct(x, batch_labels.astype(jnp.int32), batch_embed)


# speedup vs baseline: 1.1891x; 1.1891x over previous
import jax
import jax.numpy as jnp
from jax import lax
from jax.experimental import pallas as pl
from jax.experimental.pallas import tpu as pltpu
from jax.experimental.pallas import tpu_sc as plsc

EMBED_DIM = 64
NUM_BATCHES = 1000
B = 16384

NC = 2
NS = 16
NW = NC * NS
B_PER_W = B // NW
N_SUB = 4
SUB = B_PER_W // N_SUB


def _sc_body(x_hbm, idx_hbm, table_hbm, out_hbm,
             idx_v, rows_v, x_v, x_sem, g_sems, o_sem):
    wid = lax.axis_index("s") * NC + lax.axis_index("c")
    base = wid * B_PER_W

    x_copy = pltpu.async_copy(x_hbm.at[pl.ds(base, B_PER_W)], x_v, x_sem)
    pltpu.sync_copy(idx_hbm.at[pl.ds(base, B_PER_W)], idx_v)
    gathers = [None, None]
    for j in range(2):
        gathers[j] = pltpu.async_copy(
            table_hbm.at[idx_v.at[pl.ds(j * SUB, SUB)]],
            rows_v.at[j], g_sems.at[j])
    x_copy.wait()

    stores = []
    for j in range(N_SUB):
        gathers[j % 2].wait()

        def sub_row(p, _):
            r = j * SUB + p
            for c in range(EMBED_DIM // 16):
                sl = pl.ds(c * 16, 16)
                x_v[r, sl] = x_v[r, sl] - rows_v[j % 2, p, sl]
            return 0

        lax.fori_loop(0, SUB, sub_row, 0)
        stores.append(pltpu.async_copy(
            x_v.at[pl.ds(j * SUB, SUB)],
            out_hbm.at[pl.ds(base + j * SUB, SUB)], o_sem))
        if j + 2 < N_SUB:
            gathers[j % 2] = pltpu.async_copy(
                table_hbm.at[idx_v.at[pl.ds((j + 2) * SUB, SUB)]],
                rows_v.at[j % 2], g_sems.at[j % 2])
    for s in stores:
        s.wait()


@jax.jit
def _batch_correct(x, batch_labels, batch_embed):
    mesh = plsc.VectorSubcoreMesh(core_axis_name="c", subcore_axis_name="s")
    tpad = jnp.pad(batch_embed, ((0, 0), (0, 128 - EMBED_DIM)))
    return pl.kernel(
        _sc_body,
        out_type=jax.ShapeDtypeStruct((B, EMBED_DIM), jnp.float32),
        mesh=mesh,
        scratch_types=[
            pltpu.VMEM((B_PER_W,), jnp.int32),
            pltpu.VMEM((2, SUB, 128), jnp.float32),
            pltpu.VMEM((B_PER_W, EMBED_DIM), jnp.float32),
            pltpu.SemaphoreType.DMA,
            pltpu.SemaphoreType.DMA((2,)),
            pltpu.SemaphoreType.DMA,
        ],
        compiler_params=pltpu.CompilerParams(
            use_tc_tiling_on_sc=True,
            disable_bounds_checks=True,
            disable_semaphore_checks=True,
            skip_device_barrier=True,
        ),
    )(x, batch_labels, tpad)


def kernel(x, batch_labels, batch_embed):
    return _batch_correct(x, batch_labels.astype(jnp.int32), batch_embed)
